# Initial kernel scaffold; baseline (speedup 1.0000x reference)
import jax
import jax.numpy as jnp
from jax.experimental import pallas as pl


def kernel(feat_user, feat_loc, feat_price, W1, b1, W2, b2, W3, b3, theta):
    raise NotImplementedError("write your pallas kernel here")


# speedup vs baseline: 6.0564x; 6.0564x over previous
import functools

import jax
import jax.numpy as jnp
from jax import lax
from jax.experimental import pallas as pl
from jax.experimental.pallas import tpu as pltpu

B = 16384
USER_DIM = 128
LOC_DIM = 64
K = 64
EPS = 1e-08
BM = 1024
GRID = B // BM


def _body(fu, fl, fp, w1t, b1, w2t, b2, w3t, b3, amu, bmu, asd, bsd,
          out_ref, acc_ref):
    i = pl.program_id(0)

    @pl.when(i == 0)
    def _init():
        acc_ref[0] = 0.0
        acc_ref[1] = 0.0

    x = fu[...]
    h = jnp.dot(x, w1t[...], preferred_element_type=jnp.float32) + b1[...]
    h = jnp.where(h >= 0, h, 0.01 * h)
    h = jnp.dot(h, w2t[...], preferred_element_type=jnp.float32) + b2[...]
    h = jnp.where(h >= 0, h, 0.01 * h)
    logits = jnp.dot(h, w3t[...], preferred_element_type=jnp.float32) + b3[...]

    rowmax = jnp.max(logits, axis=1, keepdims=True)
    iota = lax.broadcasted_iota(jnp.int32, (BM, K), 1)
    first = jnp.min(jnp.where(logits == rowmax, iota, K), axis=1, keepdims=True)
    onehot = iota == first

    loc = fl[...]
    mu_all = jnp.dot(loc, amu[...], preferred_element_type=jnp.float32,
                     precision=lax.Precision.HIGHEST) + bmu[...]
    sd_all = jnp.dot(loc, asd[...], preferred_element_type=jnp.float32,
                     precision=lax.Precision.HIGHEST) + bsd[...]
    mu = jnp.sum(jnp.where(onehot, mu_all, 0.0), axis=1, keepdims=True)
    sd = jnp.sum(jnp.where(onehot, sd_all, 0.0), axis=1, keepdims=True)

    delta = jnp.abs(sd) + EPS
    p = fp[...]
    sq = jnp.square(mu - p) / jnp.square(delta) * 0.5
    acc_ref[0] += jnp.sum(jnp.log(delta))
    acc_ref[1] += jnp.sum(sq)

    @pl.when(i == GRID - 1)
    def _fin():
        out_ref[0, 0] = -(acc_ref[0] - acc_ref[1]) / B


@jax.jit
def _run(feat_user, feat_loc, feat_price, W1, b1, W2, b2, W3, b3, theta):
    theta_mu = theta[:, 0, :]
    theta_sd = theta[:, 1, :]
    amu = theta_mu[:, :LOC_DIM].T
    asd = theta_sd[:, :LOC_DIM].T
    bmu = theta_mu[:, LOC_DIM].reshape(1, K)
    bsd = theta_sd[:, LOC_DIM].reshape(1, K)

    row_spec = lambda cols: pl.BlockSpec((BM, cols), lambda i: (i, 0))
    full = lambda a: pl.BlockSpec(a.shape, lambda i: (0,) * a.ndim)
    args = (feat_user, feat_loc, feat_price,
            W1.T, b1.reshape(1, 32), W2.T, b2.reshape(1, 16),
            W3.T, b3.reshape(1, K), amu, bmu, asd, bsd)
    in_specs = [row_spec(USER_DIM), row_spec(LOC_DIM), row_spec(1)] + \
        [full(a) for a in args[3:]]
    out = pl.pallas_call(
        _body,
        grid=(GRID,),
        in_specs=in_specs,
        out_specs=pl.BlockSpec(memory_space=pltpu.SMEM),
        out_shape=jax.ShapeDtypeStruct((1, 1), jnp.float32),
        scratch_shapes=[pltpu.SMEM((2,), jnp.float32)],
    )(*args)
    return out[0, 0]


def kernel(feat_user, feat_loc, feat_price, W1, b1, W2, b2, W3, b3, theta):
    return _run(feat_user, feat_loc, feat_price, W1, b1, W2, b2, W3, b3, theta)


# speedup vs baseline: 7.1283x; 1.1770x over previous
import functools

import jax
import jax.numpy as jnp
from jax import lax
from jax.experimental import pallas as pl
from jax.experimental.pallas import tpu as pltpu

B = 16384
USER_DIM = 128
LOC_DIM = 64
K = 64
EPS = 1e-08
BM = 1024
GRID = B // BM


def _body(fu, fl, fp, w1t, b1, w2t, b2, w3t, b3, amu, bmu, asd, bsd,
          out_ref, acc_ref):
    i = pl.program_id(0)

    @pl.when(i == 0)
    def _init():
        acc_ref[0] = 0.0

    x = fu[...]
    h = jnp.dot(x, w1t[...], preferred_element_type=jnp.float32) + b1[...]
    h = jnp.where(h >= 0, h, 0.01 * h)
    h = jnp.dot(h, w2t[...], preferred_element_type=jnp.float32) + b2[...]
    h = jnp.where(h >= 0, h, 0.01 * h)
    logits = jnp.dot(h, w3t[...], preferred_element_type=jnp.float32) + b3[...]

    rowmax = jnp.max(logits, axis=1, keepdims=True)
    iota = lax.broadcasted_iota(jnp.int32, (BM, K), 1)
    first = jnp.min(jnp.where(logits == rowmax, iota, K), axis=1, keepdims=True)
    onehot = iota == first

    loc = fl[...]
    mu_all = jnp.dot(loc, amu[...], preferred_element_type=jnp.float32) + bmu[...]
    sd_all = jnp.dot(loc, asd[...], preferred_element_type=jnp.float32) + bsd[...]

    d = jnp.abs(sd_all) + EPS
    p = fp[...]
    r = (mu_all - p) / d
    contrib = jnp.log(d) - 0.5 * (r * r)
    acc_ref[0] += jnp.sum(jnp.where(onehot, contrib, 0.0))

    @pl.when(i == GRID - 1)
    def _fin():
        out_ref[0, 0] = -acc_ref[0] / B


@jax.jit
def _run(feat_user, feat_loc, feat_price, W1, b1, W2, b2, W3, b3, theta):
    theta_mu = theta[:, 0, :]
    theta_sd = theta[:, 1, :]
    amu = theta_mu[:, :LOC_DIM].T
    asd = theta_sd[:, :LOC_DIM].T
    bmu = theta_mu[:, LOC_DIM].reshape(1, K)
    bsd = theta_sd[:, LOC_DIM].reshape(1, K)

    row_spec = lambda cols: pl.BlockSpec((BM, cols), lambda i: (i, 0))
    full = lambda a: pl.BlockSpec(a.shape, lambda i: (0,) * a.ndim)
    args = (feat_user, feat_loc, feat_price,
            W1.T, b1.reshape(1, 32), W2.T, b2.reshape(1, 16),
            W3.T, b3.reshape(1, K), amu, bmu, asd, bsd)
    in_specs = [row_spec(USER_DIM), row_spec(LOC_DIM), row_spec(1)] + \
        [full(a) for a in args[3:]]
    out = pl.pallas_call(
        _body,
        grid=(GRID,),
        in_specs=in_specs,
        out_specs=pl.BlockSpec(memory_space=pltpu.SMEM),
        out_shape=jax.ShapeDtypeStruct((1, 1), jnp.float32),
        scratch_shapes=[pltpu.SMEM((2,), jnp.float32)],
    )(*args)
    return out[0, 0]


def kernel(feat_user, feat_loc, feat_price, W1, b1, W2, b2, W3, b3, theta):
    return _run(feat_user, feat_loc, feat_price, W1, b1, W2, b2, W3, b3, theta)
